# baseline (speedup 1.0000x reference)
import jax
import jax.numpy as jnp
from jax.experimental import pallas as pl
from jax.experimental.pallas import tpu as pltpu

_KC = 1024
_NBUF = 5


def _nnue_kernel(white_ref, black_ref, stm_ref, ftw_ref, ftb_ref,
                 l1w_ref, l1b_ref, l2w_ref, l2b_ref,
                 out_ref, wbuf, bbuf, fbuf, acc_w, acc_b, sems):
    nfeat = white_ref.shape[1]
    m = ftw_ref.shape[0]
    nk = nfeat // _KC

    def copies(k, slot):
        sl = pl.ds(k * _KC, _KC)
        return (
            pltpu.make_async_copy(white_ref.at[:, sl], wbuf.at[slot], sems.at[0, slot]),
            pltpu.make_async_copy(black_ref.at[:, sl], bbuf.at[slot], sems.at[1, slot]),
            pltpu.make_async_copy(ftw_ref.at[:, sl], fbuf.at[slot], sems.at[2, slot]),
        )

    for k in range(_NBUF):
        for c in copies(k, k):
            c.start()

    dn = (((1,), (1,)), ((), ()))
    for k in range(nk):
        slot = k % _NBUF
        for c in copies(k, slot):
            c.wait()
        wblk = wbuf[slot].astype(jnp.bfloat16)
        bblk = bbuf[slot].astype(jnp.bfloat16)
        fblk = fbuf[slot].astype(jnp.bfloat16)
        pw = jax.lax.dot_general(wblk, fblk, dn, preferred_element_type=jnp.float32)
        pb = jax.lax.dot_general(bblk, fblk, dn, preferred_element_type=jnp.float32)
        if k == 0:
            acc_w[...] = pw
            acc_b[...] = pb
        else:
            acc_w[...] += pw
            acc_b[...] += pb
        if k + _NBUF < nk:
            for c in copies(k + _NBUF, slot):
                c.start()

    w = acc_w[...] + ftb_ref[...]
    b = acc_b[...] + ftb_ref[...]
    stm = stm_ref[...]
    d = w - b
    x1 = jnp.clip(b + stm * d, 0.0, 1.0)
    x2 = jnp.clip(w - stm * d, 0.0, 1.0)
    h = jax.lax.dot_general(x1.astype(jnp.bfloat16),
                            l1w_ref[:, :m].astype(jnp.bfloat16), dn,
                            preferred_element_type=jnp.float32)
    h = h + jax.lax.dot_general(x2.astype(jnp.bfloat16),
                                l1w_ref[:, m:].astype(jnp.bfloat16), dn,
                                preferred_element_type=jnp.float32)
    h = jnp.clip(h + l1b_ref[...], 0.0, 1.0)
    out = jnp.sum(h * l2w_ref[...], axis=1, keepdims=True)
    out_ref[...] = out + l2b_ref[0, 0]


def kernel(white_features, black_features, stm, ft_w, ft_b, l1_w, l1_b, l2_w, l2_b):
    bsz, _ = white_features.shape
    m = ft_w.shape[0]
    n = l1_w.shape[0]

    return pl.pallas_call(
        _nnue_kernel,
        in_specs=[
            pl.BlockSpec(memory_space=pl.ANY),
            pl.BlockSpec(memory_space=pl.ANY),
            pl.BlockSpec(memory_space=pltpu.VMEM),
            pl.BlockSpec(memory_space=pl.ANY),
            pl.BlockSpec(memory_space=pltpu.VMEM),
            pl.BlockSpec(memory_space=pltpu.VMEM),
            pl.BlockSpec(memory_space=pltpu.VMEM),
            pl.BlockSpec(memory_space=pltpu.VMEM),
            pl.BlockSpec(memory_space=pltpu.SMEM),
        ],
        out_specs=pl.BlockSpec(memory_space=pltpu.VMEM),
        out_shape=jax.ShapeDtypeStruct((bsz, 1), jnp.float32),
        scratch_shapes=[
            pltpu.VMEM((_NBUF, bsz, _KC), jnp.float32),
            pltpu.VMEM((_NBUF, bsz, _KC), jnp.float32),
            pltpu.VMEM((_NBUF, m, _KC), jnp.float32),
            pltpu.VMEM((bsz, m), jnp.float32),
            pltpu.VMEM((bsz, m), jnp.float32),
            pltpu.SemaphoreType.DMA((3, _NBUF)),
        ],
    )(white_features, black_features, stm, ft_w, ft_b.reshape(1, m),
      l1_w, l1_b.reshape(1, n), l2_w, l2_b.reshape(1, 1))
